# baseline (speedup 1.0000x reference)
import functools

import jax
import jax.numpy as jnp
from jax.experimental import pallas as pl
from jax.experimental.pallas import tpu as pltpu

B, L, T, KH, DH = 16, 2, 2048, 8, 64
ROW = 2 * KH * DH
TB = 512


def _body(s_ref, buf_ref, kv_ref, out_ref):
    m = pl.program_id(0)
    t = pl.program_id(1)
    layer = s_ref[0]
    idx = s_ref[1]
    out_ref[...] = buf_ref[...]
    local = idx - t * TB

    @pl.when((m % L == layer) & (local >= 0) & (local < TB))
    def _():
        b = m // L
        out_ref[pl.ds(local, 1), :] = kv_ref[pl.ds(b, 1), :]


@jax.jit
def _run(buf3, kv, scalars):
    grid = (B * L, T // TB)
    return pl.pallas_call(
        _body,
        grid_spec=pltpu.PrefetchScalarGridSpec(
            num_scalar_prefetch=1,
            grid=grid,
            in_specs=[
                pl.BlockSpec((None, TB, ROW), lambda m, t, s: (m, t, 0)),
                pl.BlockSpec((B, ROW), lambda m, t, s: (0, 0)),
            ],
            out_specs=pl.BlockSpec((None, TB, ROW), lambda m, t, s: (m, t, 0)),
        ),
        out_shape=jax.ShapeDtypeStruct((B * L, T, ROW), jnp.float32),
        compiler_params=pltpu.CompilerParams(
            dimension_semantics=("parallel", "parallel"),
        ),
    )(scalars, buf3, kv)


def kernel(buf, k_step, v_step, layer, idx):
    layer = jnp.clip(jnp.asarray(layer, jnp.int32), 0, L - 1)
    idx = jnp.clip(jnp.asarray(idx, jnp.int32), 0, T - 1)
    step = jnp.clip(idx, 0, k_step.shape[1] - 1)
    ks = jax.lax.dynamic_index_in_dim(k_step, step, axis=1, keepdims=False)
    vs = jax.lax.dynamic_index_in_dim(v_step, step, axis=1, keepdims=False)
    kv = jnp.concatenate([ks.reshape(B, KH * DH), vs.reshape(B, KH * DH)], axis=1)
    scalars = jnp.stack([layer, idx])
    out3 = _run(buf.reshape(B * L, T, ROW), kv, scalars)
    return out3.reshape(B, L, T, 2, KH, DH)


# speedup vs baseline: 1.3813x; 1.3813x over previous
import jax
import jax.numpy as jnp
from jax.experimental import pallas as pl
from jax.experimental.pallas import tpu as pltpu

B, L, T, KH, DH = 16, 2, 2048, 8, 64
ROW = 2 * KH * DH


def _body(layer_ref, idx_ref, kv_ref, buf_any, out_any, sem):
    del buf_any
    layer = layer_ref[0]
    idx = idx_ref[0]
    for b in range(B):
        pltpu.make_async_copy(
            kv_ref.at[b], out_any.at[b * L + layer, idx], sem
        ).start()
    for b in range(B):
        pltpu.make_async_copy(
            kv_ref.at[b], out_any.at[b * L + layer, idx], sem
        ).wait()


@jax.jit
def _run(layer_s, idx_s, kv, buf3):
    return pl.pallas_call(
        _body,
        in_specs=[
            pl.BlockSpec(memory_space=pltpu.SMEM),
            pl.BlockSpec(memory_space=pltpu.SMEM),
            pl.BlockSpec(memory_space=pltpu.VMEM),
            pl.BlockSpec(memory_space=pl.ANY),
        ],
        out_specs=pl.BlockSpec(memory_space=pl.ANY),
        out_shape=jax.ShapeDtypeStruct((B * L, T, ROW), jnp.float32),
        scratch_shapes=[pltpu.SemaphoreType.DMA],
        input_output_aliases={3: 0},
    )(layer_s, idx_s, kv, buf3)


def kernel(buf, k_step, v_step, layer, idx):
    layer = jnp.clip(jnp.asarray(layer, jnp.int32), 0, L - 1)
    idx = jnp.clip(jnp.asarray(idx, jnp.int32), 0, T - 1)
    step = jnp.clip(idx, 0, k_step.shape[1] - 1)
    ks = jax.lax.dynamic_index_in_dim(k_step, step, axis=1, keepdims=False)
    vs = jax.lax.dynamic_index_in_dim(v_step, step, axis=1, keepdims=False)
    kv = jnp.concatenate([ks.reshape(B, KH * DH), vs.reshape(B, KH * DH)], axis=1)
    out3 = _run(layer.reshape(1), idx.reshape(1), kv, buf.reshape(B * L, T, ROW))
    return out3.reshape(B, L, T, 2, KH, DH)
